# baseline (speedup 1.0000x reference)
import functools

import jax
import jax.numpy as jnp
from jax import lax
from jax.experimental import pallas as pl
from jax.experimental.pallas import tpu as pltpu
from jax.experimental.pallas import tpu_sc as plsc

VOCAB = 1000000
N_SEG = 2
MAX_LEN = 200
EMBED = 64
BATCH = 4096
WIDE = 128
NLANE = 16

N = BATCH * MAX_LEN
NC, NS = 2, 16
NW = NC * NS
ROWS_PER_W = N // NW
CHUNK = 128
NCHUNKS = ROWS_PER_W // CHUNK

SLAB = (VOCAB // NW) // 8 * 8
TAIL = VOCAB - SLAB * NW
WCH = 168
WNCH = SLAB // WCH


def _lane_copy(dst, src, nrows, unroll=4):
  def body(r, c):
    for d in range(EMBED // NLANE):
      s = pl.ds(d * NLANE, NLANE)
      dst[r, s] = src[r, s]
    return c

  lax.fori_loop(0, nrows, body, 0, unroll=unroll)


def _widen_body(tok_hbm, wide_hbm, a0, a1, b0, b1,
                semr0, semr1, semw0, semw1):
  wid = lax.axis_index("s") * NC + lax.axis_index("c")
  base = pl.multiple_of(wid * SLAB, 8)
  A = (a0, a1)
  B = (b0, b1)
  semr = (semr0, semr1)
  semw = (semw0, semw1)

  def rd(k, p):
    pltpu.async_copy(tok_hbm.at[pl.ds(base + k * WCH, WCH)], A[p], semr[p])

  def wait_rd(k, p):
    pltpu.make_async_copy(
        tok_hbm.at[pl.ds(base + k * WCH, WCH)], A[p], semr[p]).wait()

  def wr(k, p):
    pltpu.async_copy(B[p], wide_hbm.at[pl.ds(base + k * WCH, WCH)], semw[p])

  def wait_wr(k, p):
    pltpu.make_async_copy(
        B[p], wide_hbm.at[pl.ds(base + k * WCH, WCH)], semw[p]).wait()

  rd(0, 0)

  def step(j, carry):
    for p in (0, 1):
      k = 2 * j + p

      @pl.when(k + 1 < WNCH)
      def _():
        rd(k + 1, 1 - p)

      wait_rd(k, p)

      @pl.when(k >= 2)
      def _():
        wait_wr(k - 2, p)

      _lane_copy(B[p], A[p], WCH)
      wr(k, p)
    return carry

  lax.fori_loop(0, WNCH // 2, step, 0)
  wait_wr(WNCH - 2, 0)
  wait_wr(WNCH - 1, 1)

  @pl.when(wid == 0)
  def _():
    tb = SLAB * NW
    pltpu.sync_copy(tok_hbm.at[pl.ds(tb, TAIL)], a0.at[pl.ds(0, TAIL)])
    _lane_copy(b0, a0, TAIL)
    pltpu.sync_copy(b0.at[pl.ds(0, TAIL)], wide_hbm.at[pl.ds(tb, TAIL)])


def _gather_body(wide_hbm, comb_hbm, idx_hbm, cidx_hbm, out_hbm,
                 comb_s, idx_v, cidx_v, rows0, rows1, nar0, nar1,
                 semc0, semc1, semt0, semt1, semo0, semo1):
  sid = lax.axis_index("s")
  wid = sid * NC + lax.axis_index("c")
  wbase = wid * ROWS_PER_W

  @pl.when(sid == 0)
  def _():
    pltpu.sync_copy(comb_hbm, comb_s)

  pltpu.sync_copy(idx_hbm.at[pl.ds(wbase, ROWS_PER_W)], idx_v)
  pltpu.sync_copy(cidx_hbm.at[pl.ds(wbase, ROWS_PER_W)], cidx_v)
  plsc.subcore_barrier()

  rows = (rows0, rows1)
  nar = (nar0, nar1)
  semc = (semc0, semc1)
  semt = (semt0, semt1)
  semo = (semo0, semo1)

  def tok(k, p):
    off = k * CHUNK
    pltpu.async_copy(
        wide_hbm.at[idx_v.at[pl.ds(off, CHUNK)]], rows[p], semt[p])

  def wait_tok(k, p):
    off = k * CHUNK
    pltpu.make_async_copy(
        wide_hbm.at[idx_v.at[pl.ds(off, CHUNK)]], rows[p], semt[p]).wait()

  def comb_add(k, p):
    off = k * CHUNK
    pltpu.async_copy(
        comb_s.at[cidx_v.at[pl.ds(off, CHUNK)]], rows[p], semc[p], add=True)

  def wait_comb(k, p):
    off = k * CHUNK
    pltpu.make_async_copy(
        comb_s.at[cidx_v.at[pl.ds(off, CHUNK)]], rows[p], semc[p]).wait()

  def store(k, p):
    pltpu.async_copy(
        nar[p], out_hbm.at[pl.ds(wbase + k * CHUNK, CHUNK)], semo[p])

  def wait_store(k, p):
    pltpu.make_async_copy(
        nar[p], out_hbm.at[pl.ds(wbase + k * CHUNK, CHUNK)], semo[p]).wait()

  def backend(k, p):
    wait_comb(k, p)

    @pl.when(k >= 2)
    def _():
      wait_store(k - 2, p)

    _lane_copy(nar[p], rows[p], CHUNK)
    store(k, p)

    @pl.when(k + 2 < NCHUNKS)
    def _():
      tok(k + 2, p)

  tok(0, 0)
  tok(1, 1)

  def step(j, carry):
    for p in (0, 1):
      k = 2 * j + p
      wait_tok(k, p)
      comb_add(k, p)

      @pl.when(k >= 1)
      def _():
        backend(k - 1, 1 - p)
    return carry

  lax.fori_loop(0, NCHUNKS // 2, step, 0)
  backend(NCHUNKS - 1, (NCHUNKS - 1) % 2)
  wait_store(NCHUNKS - 2, 0)
  wait_store(NCHUNKS - 1, 1)


@jax.jit
def _run(tok_table, comb, idx, cidx):
  mesh = plsc.VectorSubcoreMesh(core_axis_name="c", subcore_axis_name="s")
  widen = pl.kernel(
      _widen_body,
      out_type=jax.ShapeDtypeStruct((VOCAB, WIDE), jnp.float32),
      mesh=mesh,
      scratch_types=[
          pltpu.VMEM((WCH, EMBED), jnp.float32),
          pltpu.VMEM((WCH, EMBED), jnp.float32),
          pltpu.VMEM((WCH, WIDE), jnp.float32),
          pltpu.VMEM((WCH, WIDE), jnp.float32),
          pltpu.SemaphoreType.DMA,
          pltpu.SemaphoreType.DMA,
          pltpu.SemaphoreType.DMA,
          pltpu.SemaphoreType.DMA,
      ],
  )
  wide = widen(tok_table)
  f = pl.kernel(
      _gather_body,
      out_type=jax.ShapeDtypeStruct((N, EMBED), jnp.float32),
      mesh=mesh,
      scratch_types=[
          pltpu.VMEM_SHARED((N_SEG * MAX_LEN, WIDE), jnp.float32),
          pltpu.VMEM((ROWS_PER_W,), jnp.int32),
          pltpu.VMEM((ROWS_PER_W,), jnp.int32),
          pltpu.VMEM((CHUNK, WIDE), jnp.float32),
          pltpu.VMEM((CHUNK, WIDE), jnp.float32),
          pltpu.VMEM((CHUNK, EMBED), jnp.float32),
          pltpu.VMEM((CHUNK, EMBED), jnp.float32),
          pltpu.SemaphoreType.DMA,
          pltpu.SemaphoreType.DMA,
          pltpu.SemaphoreType.DMA,
          pltpu.SemaphoreType.DMA,
          pltpu.SemaphoreType.DMA,
          pltpu.SemaphoreType.DMA,
      ],
  )
  return f(wide, comb, idx, cidx)


def kernel(seq, seg, tok_table, seg_table, pos_table):
  comb = (seg_table[:, None, :] + pos_table[None, :, :]).reshape(
      N_SEG * MAX_LEN, EMBED)
  comb = jnp.pad(comb, ((0, 0), (0, WIDE - EMBED)))
  idx = seq.reshape(N)
  cidx = (seg * MAX_LEN + jnp.arange(MAX_LEN, dtype=jnp.int32)[None, :]
          ).reshape(N)
  out = _run(tok_table, comb, idx, cidx)
  return out.reshape(BATCH, MAX_LEN, EMBED)


# speedup vs baseline: 1.3578x; 1.3578x over previous
import functools

import jax
import jax.numpy as jnp
from jax import lax
from jax.experimental import pallas as pl
from jax.experimental.pallas import tpu as pltpu
from jax.experimental.pallas import tpu_sc as plsc

VOCAB = 1000000
N_SEG = 2
MAX_LEN = 200
EMBED = 64
BATCH = 4096

N = BATCH * MAX_LEN
NC, NS = 2, 16
NW = NC * NS
ROWS_PER_W = N // NW
CHUNK = 512
NCHUNKS = ROWS_PER_W // CHUNK


def _body(tok_hbm, comb_hbm, idx_hbm, cidx_hbm, out_hbm,
          comb_s, idx_v, cidx_v, rows0, rows1,
          semc0, semc1, semt0, semt1, semo0, semo1):
  sid = lax.axis_index("s")
  wid = sid * NC + lax.axis_index("c")
  wbase = wid * ROWS_PER_W

  @pl.when(sid == 0)
  def _():
    pltpu.sync_copy(comb_hbm, comb_s)

  pltpu.sync_copy(idx_hbm.at[pl.ds(wbase, ROWS_PER_W)], idx_v)
  pltpu.sync_copy(cidx_hbm.at[pl.ds(wbase, ROWS_PER_W)], cidx_v)
  plsc.subcore_barrier()

  rows = (rows0, rows1)
  semc = (semc0, semc1)
  semt = (semt0, semt1)
  semo = (semo0, semo1)

  def gathers(k, p):
    off = k * CHUNK
    pltpu.async_copy(
        comb_s.at[cidx_v.at[pl.ds(off, CHUNK)]], rows[p], semc[p]).wait()
    pltpu.async_copy(
        tok_hbm.at[idx_v.at[pl.ds(off, CHUNK)]], rows[p], semt[p], add=True)

  def wait_tok(k, p):
    off = k * CHUNK
    pltpu.make_async_copy(
        tok_hbm.at[idx_v.at[pl.ds(off, CHUNK)]], rows[p], semt[p]).wait()

  def store(k, p):
    pltpu.async_copy(
        rows[p], out_hbm.at[pl.ds(wbase + k * CHUNK, CHUNK)], semo[p])

  def wait_store(k, p):
    pltpu.make_async_copy(
        rows[p], out_hbm.at[pl.ds(wbase + k * CHUNK, CHUNK)], semo[p]).wait()

  gathers(0, 0)
  gathers(1, 1)

  def step(j, carry):
    for p in (0, 1):
      k = 2 * j + p
      wait_tok(k, p)
      store(k, p)

      @pl.when(j < NCHUNKS // 2 - 1)
      def _():
        wait_store(k, p)
        gathers(k + 2, p)
    return carry

  lax.fori_loop(0, NCHUNKS // 2, step, 0)
  wait_store(NCHUNKS - 2, 0)
  wait_store(NCHUNKS - 1, 1)


@jax.jit
def _run(tok_table, comb, idx, cidx):
  mesh = plsc.VectorSubcoreMesh(core_axis_name="c", subcore_axis_name="s")
  f = pl.kernel(
      _body,
      out_type=jax.ShapeDtypeStruct((N, EMBED), jnp.float32),
      mesh=mesh,
      scratch_types=[
          pltpu.VMEM_SHARED((N_SEG * MAX_LEN, EMBED), jnp.float32),
          pltpu.VMEM((ROWS_PER_W,), jnp.int32),
          pltpu.VMEM((ROWS_PER_W,), jnp.int32),
          pltpu.VMEM((CHUNK, EMBED), jnp.float32),
          pltpu.VMEM((CHUNK, EMBED), jnp.float32),
          pltpu.SemaphoreType.DMA,
          pltpu.SemaphoreType.DMA,
          pltpu.SemaphoreType.DMA,
          pltpu.SemaphoreType.DMA,
          pltpu.SemaphoreType.DMA,
          pltpu.SemaphoreType.DMA,
      ],
      compiler_params=pltpu.CompilerParams(use_tc_tiling_on_sc=False),
  )
  return f(tok_table, comb, idx, cidx)


def kernel(seq, seg, tok_table, seg_table, pos_table):
  comb = (seg_table[:, None, :] + pos_table[None, :, :]).reshape(
      N_SEG * MAX_LEN, EMBED)
  idx = seq.T.reshape(N)
  cidx = (seg.T * MAX_LEN
          + jnp.arange(MAX_LEN, dtype=jnp.int32)[:, None]).reshape(N)
  out = _run(tok_table, comb, idx, cidx)
  return out.reshape(MAX_LEN, BATCH, EMBED).transpose(1, 0, 2)


# speedup vs baseline: 1.8041x; 1.3287x over previous
import functools

import jax
import jax.numpy as jnp
from jax import lax
from jax.experimental import pallas as pl
from jax.experimental.pallas import tpu as pltpu
from jax.experimental.pallas import tpu_sc as plsc

VOCAB = 1000000
N_SEG = 2
MAX_LEN = 200
EMBED = 64
BATCH = 4096
WIDE = 128

N = BATCH * MAX_LEN
NC, NS = 2, 16
NW = NC * NS
ROWS_PER_W = N // NW
CHUNK = 512
NCHUNKS = ROWS_PER_W // CHUNK


def _body(tok_hbm, comb_hbm, idx_hbm, cidx_hbm, out_hbm,
          comb_s, idx_v, cidx_v, rows0, rows1,
          semc0, semc1, semt0, semt1, semo0, semo1):
  sid = lax.axis_index("s")
  wid = sid * NC + lax.axis_index("c")
  wbase = wid * ROWS_PER_W

  @pl.when(sid == 0)
  def _():
    pltpu.sync_copy(comb_hbm, comb_s)

  pltpu.sync_copy(idx_hbm.at[pl.ds(wbase, ROWS_PER_W)], idx_v)
  pltpu.sync_copy(cidx_hbm.at[pl.ds(wbase, ROWS_PER_W)], cidx_v)
  plsc.subcore_barrier()

  rows = (rows0, rows1)
  semc = (semc0, semc1)
  semt = (semt0, semt1)
  semo = (semo0, semo1)

  def gathers(k, p):
    off = k * CHUNK
    pltpu.async_copy(
        comb_s.at[cidx_v.at[pl.ds(off, CHUNK)]], rows[p], semc[p]).wait()
    pltpu.async_copy(
        tok_hbm.at[idx_v.at[pl.ds(off, CHUNK)]], rows[p], semt[p], add=True)

  def wait_tok(k, p):
    off = k * CHUNK
    pltpu.make_async_copy(
        tok_hbm.at[idx_v.at[pl.ds(off, CHUNK)]], rows[p], semt[p]).wait()

  def store(k, p):
    pltpu.async_copy(
        rows[p],
        out_hbm.at[pl.ds(wbase + k * CHUNK, CHUNK), pl.ds(0, EMBED)],
        semo[p])

  def wait_store(k, p):
    pltpu.make_async_copy(
        rows[p],
        out_hbm.at[pl.ds(wbase + k * CHUNK, CHUNK), pl.ds(0, EMBED)],
        semo[p]).wait()

  gathers(0, 0)
  gathers(1, 1)

  def step(j, carry):
    for p in (0, 1):
      k = 2 * j + p
      wait_tok(k, p)
      store(k, p)

      @pl.when(j < NCHUNKS // 2 - 1)
      def _():
        wait_store(k, p)
        gathers(k + 2, p)
    return carry

  lax.fori_loop(0, NCHUNKS // 2, step, 0)
  wait_store(NCHUNKS - 2, 0)
  wait_store(NCHUNKS - 1, 1)


@jax.jit
def _run(tok_table, comb, idx, cidx):
  mesh = plsc.VectorSubcoreMesh(core_axis_name="c", subcore_axis_name="s")
  f = pl.kernel(
      _body,
      out_type=jax.ShapeDtypeStruct((N, WIDE), jnp.float32),
      mesh=mesh,
      scratch_types=[
          pltpu.VMEM_SHARED((N_SEG * MAX_LEN, EMBED), jnp.float32),
          pltpu.VMEM((ROWS_PER_W,), jnp.int32),
          pltpu.VMEM((ROWS_PER_W,), jnp.int32),
          pltpu.VMEM((CHUNK, EMBED), jnp.float32),
          pltpu.VMEM((CHUNK, EMBED), jnp.float32),
          pltpu.SemaphoreType.DMA,
          pltpu.SemaphoreType.DMA,
          pltpu.SemaphoreType.DMA,
          pltpu.SemaphoreType.DMA,
          pltpu.SemaphoreType.DMA,
          pltpu.SemaphoreType.DMA,
      ],
      compiler_params=pltpu.CompilerParams(use_tc_tiling_on_sc=False),
  )
  return f(tok_table, comb, idx, cidx)


def kernel(seq, seg, tok_table, seg_table, pos_table):
  comb = (seg_table[:, None, :] + pos_table[None, :, :]).reshape(
      N_SEG * MAX_LEN, EMBED)
  idx = seq.T.reshape(N)
  cidx = (seg.T * MAX_LEN
          + jnp.arange(MAX_LEN, dtype=jnp.int32)[:, None]).reshape(N)
  out = _run(tok_table, comb, idx, cidx)
  return (out[:, :EMBED].reshape(MAX_LEN, BATCH, EMBED).transpose(1, 0, 2))
